# baseline (speedup 1.0000x reference)
import jax
import jax.numpy as jnp
from jax import lax
from jax.experimental import pallas as pl
from jax.experimental.pallas import tpu as pltpu
from jax.experimental.pallas import tpu_sc as plsc

BATCH = 16384
DIM = 32
_info = plsc.get_sparse_core_info()
_NC, _NS, _L = _info.num_cores, _info.num_subcores, _info.num_lanes
_NW = _NC * _NS
_BPW = BATCH // _NW


def _mf_kernel(ui_hbm, ii_hbm, eu_hbm, ei_hbm, out_hbm,
               idx_u, idx_i, rows_u, rows_i, prod_t, out_v, sem_u, sem_i):
    wid = lax.axis_index("s") * _NC + lax.axis_index("c")
    base = wid * _BPW
    pltpu.sync_copy(ui_hbm.at[pl.ds(base, _BPW)], idx_u)
    pltpu.sync_copy(ii_hbm.at[pl.ds(base, _BPW)], idx_i)
    cp_u = pltpu.async_copy(eu_hbm.at[idx_u], rows_u, sem_u)
    cp_i = pltpu.async_copy(ei_hbm.at[idx_i], rows_i, sem_i)
    cp_u.wait()
    cp_i.wait()

    lane = lax.iota(jnp.int32, _L)

    def row_body(r, _):
        u_lo = rows_u[r, pl.ds(0, _L)]
        u_hi = rows_u[r, pl.ds(_L, _L)]
        i_lo = rows_i[r, pl.ds(0, _L)]
        i_hi = rows_i[r, pl.ds(_L, _L)]
        s = u_lo * i_lo + u_hi * i_hi
        plsc.store_scatter(prod_t, [lane * (_BPW + 1) + r], s)
        return 0

    lax.fori_loop(0, _BPW, row_body, 0, unroll=4)

    def chunk_body(c, _):
        c0 = c * _L
        acc = prod_t[pl.ds(c0, _L)]
        for l in range(1, _L):
            acc = acc + prod_t[pl.ds(l * (_BPW + 1) + c0, _L)]
        out_v[pl.ds(c0, _L)] = acc
        return 0

    lax.fori_loop(0, _BPW // _L, chunk_body, 0, unroll=2)
    pltpu.sync_copy(out_v, out_hbm.at[pl.ds(base, _BPW)])


@jax.jit
def kernel(user_indices, item_indices, embedding_user, embedding_item):
    mesh = plsc.VectorSubcoreMesh(core_axis_name="c", subcore_axis_name="s")
    run = pl.kernel(
        _mf_kernel,
        mesh=mesh,
        out_type=jax.ShapeDtypeStruct((BATCH,), jnp.float32),
        scratch_types=[
            pltpu.VMEM((_BPW,), jnp.int32),
            pltpu.VMEM((_BPW,), jnp.int32),
            pltpu.VMEM((_BPW, DIM), jnp.float32),
            pltpu.VMEM((_BPW, DIM), jnp.float32),
            pltpu.VMEM((_L * (_BPW + 1),), jnp.float32),
            pltpu.VMEM((_BPW,), jnp.float32),
            pltpu.SemaphoreType.DMA,
            pltpu.SemaphoreType.DMA,
        ],
        compiler_params=pltpu.CompilerParams(
            needs_layout_passes=False, use_tc_tiling_on_sc=False),
    )
    return run(user_indices.astype(jnp.int32), item_indices.astype(jnp.int32),
               embedding_user, embedding_item)
